# baseline (speedup 1.0000x reference)
import functools

import jax
import jax.numpy as jnp
from jax import lax
from jax.experimental import pallas as pl
from jax.experimental.pallas import tpu as pltpu
from jax.experimental.pallas import tpu_sc as plsc

NUM_FEATURES = 16
SEC = 8

_info = plsc.get_sparse_core_info()
_NC, _NS = _info.num_cores, _info.num_subcores


@functools.lru_cache(maxsize=None)
def _make_gather(s_dim: int, b_dim: int, vocab: int, chunk: int):
    half = b_dim // _NC
    k_per_s = half // chunk
    n_full = (s_dim // SEC) // 2 * 2
    tail_rows = s_dim - n_full * SEC
    c_per_sec = SEC * k_per_s
    assert half % chunk == 0 and chunk % 16 == 0 and c_per_sec % 2 == 0
    assert 0 < tail_rows and (n_full * SEC * k_per_s) % 2 == 0
    mesh = plsc.VectorSubcoreMesh(core_axis_name="c", subcore_axis_name="s")

    @functools.partial(
        pl.kernel,
        out_type=jax.ShapeDtypeStruct((s_dim, NUM_FEATURES, b_dim), jnp.float32),
        mesh=mesh,
        scratch_types=[
            [pltpu.VMEM_SHARED((SEC, half), jnp.int32)] * 2,
            pltpu.VMEM((vocab,), jnp.float32),
            [pltpu.VMEM((chunk,), jnp.int32)] * 2,
            [pltpu.VMEM((chunk,), jnp.float32)] * 2,
            pltpu.SemaphoreType.DMA,
            [pltpu.SemaphoreType.DMA] * 2,
        ],
        compiler_params=pltpu.CompilerParams(
            use_tc_tiling_on_sc=True, needs_layout_passes=False
        ),
    )
    def gather(
        sp_hbm, emb_hbm, out_hbm, sp_sh, row_v, idx_v, out_v, stsem, osem
    ):
        f = lax.axis_index("s")
        b0 = lax.axis_index("c") * half

        def stage(sec, q):
            pltpu.async_copy(
                sp_hbm.at[pl.ds(sec * SEC, SEC), pl.ds(b0, half)],
                sp_sh[q],
                stsem,
            )

        def stage_wait(q):
            pltpu.make_async_copy(
                sp_hbm.at[pl.ds(0, SEC), pl.ds(b0, half)], sp_sh[q], stsem
            ).wait()

        def stage_tail(q):
            pltpu.async_copy(
                sp_hbm.at[pl.ds(n_full * SEC, tail_rows), pl.ds(b0, half)],
                sp_sh[q].at[pl.ds(0, tail_rows)],
                stsem,
            )

        def stage_tail_wait(q):
            pltpu.make_async_copy(
                sp_hbm.at[pl.ds(0, tail_rows), pl.ds(b0, half)],
                sp_sh[q].at[pl.ds(0, tail_rows)],
                stsem,
            ).wait()

        @pl.when(f == 0)
        def _():
            stage(0, 0)

        pltpu.sync_copy(emb_hbm.at[f], row_v)

        @pl.when(f == 0)
        def _():
            stage_wait(0)
            stage(1, 1)

        plsc.subcore_barrier()

        def consume(sec, q, rows):
            for i in range(rows * k_per_s):
                bb = i % 2
                s_loc, k = divmod(i, k_per_s)
                pltpu.sync_copy(
                    sp_sh[q].at[s_loc, pl.ds(k * chunk, chunk)], idx_v[bb]
                )
                if i >= 2:
                    pltpu.make_async_copy(
                        out_v[bb], out_hbm.at[0, 0, pl.ds(0, chunk)], osem[bb]
                    ).wait()
                else:

                    @pl.when(sec > 0)
                    def _():
                        pltpu.make_async_copy(
                            out_v[bb],
                            out_hbm.at[0, 0, pl.ds(0, chunk)],
                            osem[bb],
                        ).wait()

                @plsc.parallel_loop(0, chunk, step=16, unroll=8)
                def _(j):
                    idx = idx_v[bb][pl.ds(j, 16)]
                    out_v[bb][pl.ds(j, 16)] = plsc.load_gather(row_v, [idx])

                pltpu.async_copy(
                    out_v[bb],
                    out_hbm.at[
                        sec * SEC + s_loc, f, pl.ds(b0 + k * chunk, chunk)
                    ],
                    osem[bb],
                )

        @pl.loop(0, n_full, step=2)
        def _(ss):
            for q in range(2):
                sec = ss + q
                consume(sec, q, SEC)

                @pl.when(f == 0)
                def _():
                    @pl.when(sec + 1 < n_full)
                    def _():
                        stage_wait(1 - q)

                    @pl.when(sec + 1 == n_full)
                    def _():
                        stage_tail_wait(1 - q)

                plsc.subcore_barrier()

                @pl.when(f == 0)
                def _():
                    @pl.when(sec + 2 < n_full)
                    def _():
                        stage(sec + 2, q)

                    @pl.when(sec + 2 == n_full)
                    def _():
                        stage_tail(q)

        consume(jnp.int32(n_full), 0, tail_rows)

        for bb in range(2):
            pltpu.make_async_copy(
                out_v[bb], out_hbm.at[0, 0, pl.ds(0, chunk)], osem[bb]
            ).wait()

    return gather


def kernel(species, embedding):
    b_dim, s_dim = species.shape
    vocab, feat = embedding.shape
    sp_t = species.T.astype(jnp.int32)
    emb_t = embedding.T
    out_t = _make_gather(s_dim, b_dim, vocab, 4096)(sp_t, emb_t)
    return jnp.transpose(out_t, (2, 0, 1))


# speedup vs baseline: 1.0712x; 1.0712x over previous
import functools

import jax
import jax.numpy as jnp
from jax import lax
from jax.experimental import pallas as pl
from jax.experimental.pallas import tpu as pltpu
from jax.experimental.pallas import tpu_sc as plsc

NUM_FEATURES = 16
SLAB = 8

_info = plsc.get_sparse_core_info()
_NC, _NS = _info.num_cores, _info.num_subcores


@functools.lru_cache(maxsize=None)
def _make_gather(s_dim: int, b_dim: int, vocab: int, chunk: int):
    half = b_dim // _NC
    k_per_s = half // chunk
    n_slabs = s_dim // SLAB
    tail0 = n_slabs * SLAB
    n_units = n_slabs * k_per_s
    assert half % chunk == 0 and chunk % 128 == 0 and n_units % 2 == 0
    mesh = plsc.VectorSubcoreMesh(core_axis_name="c", subcore_axis_name="s")

    @functools.partial(
        pl.kernel,
        out_type=jax.ShapeDtypeStruct((s_dim, NUM_FEATURES, b_dim), jnp.float32),
        mesh=mesh,
        scratch_types=[
            pltpu.VMEM((vocab,), jnp.float32),
            [pltpu.VMEM((SLAB, chunk), jnp.int32)] * 2,
            [pltpu.VMEM((chunk,), jnp.int32)] * 2,
            [pltpu.VMEM((chunk,), jnp.float32)] * 2,
            [pltpu.SemaphoreType.DMA] * 2,
            [pltpu.SemaphoreType.DMA] * 2,
        ],
        compiler_params=pltpu.CompilerParams(
            use_tc_tiling_on_sc=True, needs_layout_passes=False
        ),
    )
    def gather(
        sp_hbm, emb_hbm, out_hbm, row_v, slab_v, tidx_v, out_v, isem, osem
    ):
        f = lax.axis_index("s")
        b0 = lax.axis_index("c") * half

        pltpu.sync_copy(emb_hbm.at[f], row_v)

        def slab_src(u):
            g = u // k_per_s
            k = lax.rem(u, k_per_s)
            return sp_hbm.at[
                pl.ds(g * SLAB, SLAB), pl.ds(b0 + k * chunk, chunk)
            ]

        for b in range(2):
            pltpu.async_copy(slab_src(b), slab_v[b], isem[b])

        @pl.loop(0, n_units, step=2)
        def _(uu):
            for b in range(2):
                u = uu + b
                g = u // k_per_s
                k = lax.rem(u, k_per_s)
                pltpu.make_async_copy(slab_src(u), slab_v[b], isem[b]).wait()
                for r in range(SLAB):
                    bb = r % 2
                    if r >= 2 or b == 1:
                        pltpu.make_async_copy(
                            out_v[bb],
                            out_hbm.at[0, 0, pl.ds(0, chunk)],
                            osem[bb],
                        ).wait()
                    else:

                        @pl.when(uu > 0)
                        def _():
                            pltpu.make_async_copy(
                                out_v[bb],
                                out_hbm.at[0, 0, pl.ds(0, chunk)],
                                osem[bb],
                            ).wait()

                    @plsc.parallel_loop(0, chunk, step=16, unroll=8)
                    def _(j):
                        idx = slab_v[b][r, pl.ds(j, 16)]
                        out_v[bb][pl.ds(j, 16)] = plsc.load_gather(
                            row_v, [idx]
                        )

                    pltpu.async_copy(
                        out_v[bb],
                        out_hbm.at[
                            g * SLAB + r, f, pl.ds(b0 + k * chunk, chunk)
                        ],
                        osem[bb],
                    )

                @pl.when(u + 2 < n_units)
                def _():
                    pltpu.async_copy(slab_src(u + 2), slab_v[b], isem[b])

        n_tail = (s_dim - tail0) * k_per_s
        for i in range(n_tail):
            bb = i % 2
            s, k = divmod(i, k_per_s)
            pltpu.sync_copy(
                sp_hbm.at[tail0 + s, pl.ds(b0 + k * chunk, chunk)], tidx_v[bb]
            )
            pltpu.make_async_copy(
                out_v[bb], out_hbm.at[0, 0, pl.ds(0, chunk)], osem[bb]
            ).wait()

            @plsc.parallel_loop(0, chunk, step=16, unroll=8)
            def _(j):
                idx = tidx_v[bb][pl.ds(j, 16)]
                out_v[bb][pl.ds(j, 16)] = plsc.load_gather(row_v, [idx])

            pltpu.async_copy(
                out_v[bb],
                out_hbm.at[tail0 + s, f, pl.ds(b0 + k * chunk, chunk)],
                osem[bb],
            )

        for bb in range(2):
            pltpu.make_async_copy(
                out_v[bb], out_hbm.at[0, 0, pl.ds(0, chunk)], osem[bb]
            ).wait()

    return gather


def kernel(species, embedding):
    b_dim, s_dim = species.shape
    vocab, feat = embedding.shape
    sp_t = species.T.astype(jnp.int32)
    emb_t = embedding.T
    out_t = _make_gather(s_dim, b_dim, vocab, 1024)(sp_t, emb_t)
    return jnp.transpose(out_t, (2, 0, 1))
